# baseline (speedup 1.0000x reference)
import functools

import numpy as np
import jax
import jax.numpy as jnp
from jax import lax
from jax.experimental import pallas as pl
from jax.experimental.pallas import tpu as pltpu
from jax.experimental.pallas import tpu_sc as plsc

NC = 2
NS = 16
NW = NC * NS
LANES = 16
CH = 128
NBUF = 8
G = 64
EPS = 1e-5
DEG_W = 16


def _zero_vmem(buf, n_rows, n_cols):
    z = jnp.zeros((LANES,), jnp.float32)
    for i in range(n_rows):
        for j in range(n_cols // LANES):
            buf[i, pl.ds(j * LANES, LANES)] = z


def _fill_ones(buf, n_rows, n_cols):
    o = jnp.ones((LANES,), jnp.float32)
    for i in range(n_rows):
        for j in range(n_cols // LANES):
            buf[i, pl.ds(j * LANES, LANES)] = o


def _make_hist_kernel(n_pad, n_chunks):
    rows_per_sub = n_pad // NS
    z_rows = 64
    wave = 8
    mesh = plsc.VectorSubcoreMesh(core_axis_name="c", subcore_axis_name="s")

    @functools.partial(
        pl.kernel,
        out_type=jax.ShapeDtypeStruct((2 * n_pad, DEG_W), jnp.float32),
        mesh=mesh,
        scratch_types=[
            pltpu.VMEM_SHARED((n_pad, DEG_W), jnp.float32),
            pltpu.VMEM((n_chunks, CH), jnp.int32),
            pltpu.VMEM((CH, DEG_W), jnp.float32),
            pltpu.VMEM((z_rows, DEG_W), jnp.float32),
            pltpu.SemaphoreType.DMA,
        ],
        compiler_params=pltpu.CompilerParams(use_tc_tiling_on_sc=False),
    )
    def k(dst_hbm, out_hbm, acc, didx, ones, zbuf, sem):
        c = lax.axis_index("c")
        s = lax.axis_index("s")
        wid = s * NC + c
        pltpu.sync_copy(dst_hbm.at[wid], didx)
        _fill_ones(ones, CH, DEG_W)
        _zero_vmem(zbuf, z_rows, DEG_W)
        r0 = s * rows_per_sub
        for t in range(rows_per_sub // z_rows):
            pltpu.sync_copy(zbuf, acc.at[pl.ds(r0 + t * z_rows, z_rows)])
        plsc.subcore_barrier()

        def body(i, carry):
            descs = []
            for j in range(wave):
                t = i * wave + j
                descs.append(
                    pltpu.async_copy(ones, acc.at[didx.at[t]], sem, add=True))
            for d in descs:
                d.wait()
            return carry

        lax.fori_loop(0, n_chunks // wave, body, 0)
        plsc.subcore_barrier()
        out0 = c * n_pad + s * rows_per_sub
        pltpu.sync_copy(acc.at[pl.ds(r0, rows_per_sub)],
                        out_hbm.at[pl.ds(out0, rows_per_sub)])

    return k


def _make_msg_kernel(n_pad, n_chunks, h_dim):
    rows_per_sub = n_pad // NS
    z_rows = 64
    mesh = plsc.VectorSubcoreMesh(core_axis_name="c", subcore_axis_name="s")

    @functools.partial(
        pl.kernel,
        out_type=jax.ShapeDtypeStruct((2 * n_pad, h_dim), jnp.float32),
        mesh=mesh,
        scratch_types=[
            pltpu.VMEM_SHARED((n_pad, h_dim), jnp.float32),
            pltpu.VMEM((n_chunks, CH), jnp.int32),
            pltpu.VMEM((n_chunks, CH), jnp.int32),
            [pltpu.VMEM((CH, h_dim), jnp.float32) for _ in range(NBUF)],
            pltpu.VMEM((z_rows, h_dim), jnp.float32),
            [pltpu.SemaphoreType.DMA for _ in range(NBUF)],
            [pltpu.SemaphoreType.DMA for _ in range(NBUF)],
        ],
        compiler_params=pltpu.CompilerParams(use_tc_tiling_on_sc=False),
    )
    def k(h_hbm, src_hbm, dst_hbm, out_hbm, acc, sidx, didx, rows, zbuf,
          sem_g, sem_s):
        c = lax.axis_index("c")
        s = lax.axis_index("s")
        wid = s * NC + c
        pltpu.sync_copy(src_hbm.at[wid], sidx)
        pltpu.sync_copy(dst_hbm.at[wid], didx)
        _zero_vmem(zbuf, z_rows, h_dim)
        r0 = s * rows_per_sub
        for t in range(rows_per_sub // z_rows):
            pltpu.sync_copy(zbuf, acc.at[pl.ds(r0 + t * z_rows, z_rows)])
        plsc.subcore_barrier()

        def body(i, carry):
            gathers = []
            for j in range(NBUF):
                t = i * NBUF + j

                @pl.when(i > 0)
                def _drain():
                    pltpu.make_async_copy(rows[j], acc.at[didx.at[t]],
                                          sem_s[j]).wait()

                gathers.append(
                    pltpu.async_copy(h_hbm.at[sidx.at[t]], rows[j],
                                     sem_g[j]))
            for j in range(NBUF):
                t = i * NBUF + j
                gathers[j].wait()
                pltpu.async_copy(rows[j], acc.at[didx.at[t]], sem_s[j],
                                 add=True)
            return carry

        n_iter = n_chunks // NBUF
        lax.fori_loop(0, n_iter, body, 0)
        for j in range(NBUF):
            t = (n_iter - 1) * NBUF + j
            pltpu.make_async_copy(rows[j], acc.at[didx.at[t]],
                                  sem_s[j]).wait()
        plsc.subcore_barrier()
        out0 = c * n_pad + s * rows_per_sub
        pltpu.sync_copy(acc.at[pl.ds(r0, rows_per_sub)],
                        out_hbm.at[pl.ds(out0, rows_per_sub)])

    return k


def kernel(x, e, b, W1, b1, g1, be1, W2, b2, g2, be2):
    n, d = x.shape
    h_dim = W1.shape[1]
    n_edges = e.shape[1]

    n_pad = ((n + NW * 64) // (NW * 64)) * (NW * 64)
    grp = NW * CH * NBUF
    ep = ((n_edges + grp - 1) // grp) * grp
    n_chunks = ep // (NW * CH)

    src = e[0].astype(jnp.int32)
    dst = e[1].astype(jnp.int32)
    pad = ep - n_edges
    pad_np = np.arange(pad, dtype=np.int32)
    src_p = jnp.concatenate([src, jnp.asarray(pad_np % n)])
    dst_p = jnp.concatenate([dst, jnp.asarray(n + pad_np % (n_pad - n))])
    src_p = src_p.reshape(NW, n_chunks, CH)
    dst_p = dst_p.reshape(NW, n_chunks, CH)

    def mm1_body(x_ref, w_ref, o_ref):
        o_ref[...] = jnp.dot(x_ref[...], w_ref[...],
                             preferred_element_type=jnp.float32)

    h1_raw = pl.pallas_call(
        mm1_body,
        out_shape=jax.ShapeDtypeStruct((n, h_dim), jnp.float32),
    )(x, W1)

    hist = _make_hist_kernel(n_pad, n_chunks)(dst_p)

    def scale_body(hist_ref, h_ref, o_ref, dinv_ref):
        deg = hist_ref[0:n_pad, :] + hist_ref[n_pad:2 * n_pad, :] + 1.0
        dv = lax.rsqrt(deg)
        dinv_ref[...] = dv
        o_ref[...] = h_ref[...] * dv[0:n, 0:1]

    h1, dinv = pl.pallas_call(
        scale_body,
        out_shape=[jax.ShapeDtypeStruct((n, h_dim), jnp.float32),
                   jax.ShapeDtypeStruct((n_pad, DEG_W), jnp.float32)],
    )(hist, h1_raw)

    msg = _make_msg_kernel(n_pad, n_chunks, h_dim)

    p1 = msg(h1, src_p, dst_p)

    def mid_body(p_ref, h_ref, dinv_ref, b1_ref, g1_ref, be1_ref, w2_ref,
                 o_ref):
        dv = dinv_ref[0:n, 0:1]
        s = p_ref[0:n, :] + p_ref[n_pad:n_pad + n, :] + h_ref[...]
        o1 = s * dv + b1_ref[...]
        mu = jnp.mean(o1, axis=0, keepdims=True)
        var = jnp.mean((o1 - mu) ** 2, axis=0, keepdims=True)
        y = (o1 - mu) * lax.rsqrt(var + EPS) * g1_ref[...] + be1_ref[...]
        y = jnp.maximum(y, 0.0)
        h2 = jnp.dot(y, w2_ref[...], preferred_element_type=jnp.float32)
        o_ref[...] = h2 * dv

    h2 = pl.pallas_call(
        mid_body,
        out_shape=jax.ShapeDtypeStruct((n, h_dim), jnp.float32),
    )(p1, h1, dinv, b1.reshape(1, h_dim), g1.reshape(1, h_dim),
      be1.reshape(1, h_dim), W2)

    p2 = msg(h2, src_p, dst_p)

    bid = b.astype(jnp.int32).reshape(1, n)

    def out_body(p_ref, h_ref, dinv_ref, b2_ref, g2_ref, be2_ref, bid_ref,
                 o_ref):
        dv = dinv_ref[0:n, 0:1]
        s = p_ref[0:n, :] + p_ref[n_pad:n_pad + n, :] + h_ref[...]
        o2 = s * dv + b2_ref[...]
        mu = jnp.mean(o2, axis=0, keepdims=True)
        var = jnp.mean((o2 - mu) ** 2, axis=0, keepdims=True)
        y = (o2 - mu) * lax.rsqrt(var + EPS) * g2_ref[...] + be2_ref[...]
        y = jnp.maximum(y, 0.0)
        gi = lax.broadcasted_iota(jnp.int32, (G, n), 0)
        m = (bid_ref[...] == gi).astype(jnp.float32)
        sums = jnp.dot(m, y, preferred_element_type=jnp.float32)
        counts = jnp.sum(m, axis=1, keepdims=True)
        o_ref[...] = sums / jnp.maximum(counts, 1.0)

    out = pl.pallas_call(
        out_body,
        out_shape=jax.ShapeDtypeStruct((G, h_dim), jnp.float32),
    )(p2, h2, dinv, b2.reshape(1, h_dim), g2.reshape(1, h_dim),
      be2.reshape(1, h_dim), bid)

    return out


# speedup vs baseline: 1.0116x; 1.0116x over previous
import functools

import numpy as np
import jax
import jax.numpy as jnp
from jax import lax
from jax.experimental import pallas as pl
from jax.experimental.pallas import tpu as pltpu
from jax.experimental.pallas import tpu_sc as plsc

NC = 2
NS = 16
NW = NC * NS
LANES = 16
CH = 128
NBUF = 8
G = 64
EPS = 1e-5
DEG_W = 16


def _zero_vmem(buf, n_rows, n_cols):
    z = jnp.zeros((LANES,), jnp.float32)
    for i in range(n_rows):
        for j in range(n_cols // LANES):
            buf[i, pl.ds(j * LANES, LANES)] = z


def _fill_ones(buf, n_rows, n_cols):
    o = jnp.ones((LANES,), jnp.float32)
    for i in range(n_rows):
        for j in range(n_cols // LANES):
            buf[i, pl.ds(j * LANES, LANES)] = o


def _make_hist_kernel(n_pad, n_chunks):
    rows_per_sub = n_pad // NS
    z_rows = 64
    wave = 8
    mesh = plsc.VectorSubcoreMesh(core_axis_name="c", subcore_axis_name="s")

    @functools.partial(
        pl.kernel,
        out_type=jax.ShapeDtypeStruct((2 * n_pad, DEG_W), jnp.float32),
        mesh=mesh,
        scratch_types=[
            pltpu.VMEM_SHARED((n_pad, DEG_W), jnp.float32),
            pltpu.VMEM((n_chunks, CH), jnp.int32),
            pltpu.VMEM((CH, DEG_W), jnp.float32),
            pltpu.VMEM((z_rows, DEG_W), jnp.float32),
            pltpu.SemaphoreType.DMA,
        ],
        compiler_params=pltpu.CompilerParams(use_tc_tiling_on_sc=False),
    )
    def k(dst_hbm, out_hbm, acc, didx, ones, zbuf, sem):
        c = lax.axis_index("c")
        s = lax.axis_index("s")
        wid = s * NC + c
        pltpu.sync_copy(dst_hbm.at[wid], didx)
        _fill_ones(ones, CH, DEG_W)
        _zero_vmem(zbuf, z_rows, DEG_W)
        r0 = s * rows_per_sub
        for t in range(rows_per_sub // z_rows):
            pltpu.sync_copy(zbuf, acc.at[pl.ds(r0 + t * z_rows, z_rows)])
        plsc.subcore_barrier()

        def body(i, carry):
            descs = []
            for j in range(wave):
                t = i * wave + j
                descs.append(
                    pltpu.async_copy(ones, acc.at[didx.at[t]], sem, add=True))
            for d in descs:
                d.wait()
            return carry

        lax.fori_loop(0, n_chunks // wave, body, 0)
        plsc.subcore_barrier()
        out0 = c * n_pad + s * rows_per_sub
        pltpu.sync_copy(acc.at[pl.ds(r0, rows_per_sub)],
                        out_hbm.at[pl.ds(out0, rows_per_sub)])

    return k


def _make_msg_kernel(n_pad, n_chunks, h_dim):
    rows_per_sub = n_pad // NS
    z_rows = 64
    mesh = plsc.VectorSubcoreMesh(core_axis_name="c", subcore_axis_name="s")

    @functools.partial(
        pl.kernel,
        out_type=jax.ShapeDtypeStruct((2 * n_pad, h_dim), jnp.float32),
        mesh=mesh,
        scratch_types=[
            pltpu.VMEM_SHARED((n_pad, h_dim), jnp.float32),
            pltpu.VMEM((n_chunks, CH), jnp.int32),
            pltpu.VMEM((n_chunks, CH), jnp.int32),
            [pltpu.VMEM((CH, h_dim), jnp.float32) for _ in range(NBUF)],
            pltpu.VMEM((z_rows, h_dim), jnp.float32),
            [pltpu.SemaphoreType.DMA for _ in range(NBUF)],
            [pltpu.SemaphoreType.DMA for _ in range(NBUF)],
        ],
        compiler_params=pltpu.CompilerParams(use_tc_tiling_on_sc=False),
    )
    def k(h_hbm, src_hbm, dst_hbm, out_hbm, acc, sidx, didx, rows, zbuf,
          sem_g, sem_s):
        c = lax.axis_index("c")
        s = lax.axis_index("s")
        wid = s * NC + c
        pltpu.sync_copy(src_hbm.at[wid], sidx)
        pltpu.sync_copy(dst_hbm.at[wid], didx)
        _zero_vmem(zbuf, z_rows, h_dim)
        r0 = s * rows_per_sub
        for t in range(rows_per_sub // z_rows):
            pltpu.sync_copy(zbuf, acc.at[pl.ds(r0 + t * z_rows, z_rows)])
        plsc.subcore_barrier()

        def body(i, carry):
            gathers = []
            for j in range(NBUF):
                t = i * NBUF + j

                @pl.when(i > 0)
                def _drain():
                    pltpu.make_async_copy(rows[j], acc.at[didx.at[t]],
                                          sem_s[j]).wait()

                gathers.append(
                    pltpu.async_copy(h_hbm.at[sidx.at[t]], rows[j],
                                     sem_g[j]))
            for j in range(NBUF):
                t = i * NBUF + j
                gathers[j].wait()
                pltpu.async_copy(rows[j], acc.at[didx.at[t]], sem_s[j],
                                 add=True)
            return carry

        n_iter = n_chunks // NBUF
        lax.fori_loop(0, n_iter, body, 0)
        for j in range(NBUF):
            t = (n_iter - 1) * NBUF + j
            pltpu.make_async_copy(rows[j], acc.at[didx.at[t]],
                                  sem_s[j]).wait()
        plsc.subcore_barrier()
        out0 = c * n_pad + s * rows_per_sub
        pltpu.sync_copy(acc.at[pl.ds(r0, rows_per_sub)],
                        out_hbm.at[pl.ds(out0, rows_per_sub)])

    return k


def kernel(x, e, b, W1, b1, g1, be1, W2, b2, g2, be2):
    n, d = x.shape
    h_dim = W1.shape[1]
    n_edges = e.shape[1]

    n_pad = ((n + NW * 64) // (NW * 64)) * (NW * 64)
    grp = NW * CH * NBUF
    ep = ((n_edges + grp - 1) // grp) * grp
    n_chunks = ep // (NW * CH)

    ef = e.astype(jnp.int32).reshape(2 * n_edges)
    src = ef[:n_edges]
    dst = ef[n_edges:]
    pad = ep - n_edges
    pad_np = np.arange(pad, dtype=np.int32)
    src_p = jnp.concatenate([src, jnp.asarray(pad_np % n)])
    dst_p = jnp.concatenate([dst, jnp.asarray(n + pad_np % (n_pad - n))])
    src_p = src_p.reshape(NW, n_chunks, CH)
    dst_p = dst_p.reshape(NW, n_chunks, CH)

    def mm1_body(x_ref, w_ref, o_ref):
        o_ref[...] = jnp.dot(x_ref[...], w_ref[...],
                             preferred_element_type=jnp.float32)

    h1_raw = pl.pallas_call(
        mm1_body,
        out_shape=jax.ShapeDtypeStruct((n, h_dim), jnp.float32),
    )(x, W1)

    hist = _make_hist_kernel(n_pad, n_chunks)(dst_p)

    def scale_body(hist_ref, h_ref, o_ref, dinv_ref):
        deg = hist_ref[0:n, :] + hist_ref[n_pad:n_pad + n, :] + 1.0
        dv = lax.rsqrt(deg)[:, 0:1]
        dv64 = jnp.broadcast_to(dv, (n, h_dim))
        dinv_ref[...] = dv64
        o_ref[...] = h_ref[...] * dv64

    h1, dinv = pl.pallas_call(
        scale_body,
        out_shape=[jax.ShapeDtypeStruct((n, h_dim), jnp.float32),
                   jax.ShapeDtypeStruct((n, h_dim), jnp.float32)],
    )(hist, h1_raw)

    msg = _make_msg_kernel(n_pad, n_chunks, h_dim)

    p1 = msg(h1, src_p, dst_p)

    def mid_body(p_ref, h_ref, dinv_ref, b1_ref, g1_ref, be1_ref, w2_ref,
                 o_ref):
        dv = dinv_ref[...]
        s = p_ref[0:n, :] + p_ref[n_pad:n_pad + n, :] + h_ref[...]
        o1 = s * dv + b1_ref[...]
        mu = jnp.mean(o1, axis=0, keepdims=True)
        var = jnp.mean((o1 - mu) ** 2, axis=0, keepdims=True)
        y = (o1 - mu) * lax.rsqrt(var + EPS) * g1_ref[...] + be1_ref[...]
        y = jnp.maximum(y, 0.0)
        h2 = jnp.dot(y, w2_ref[...], preferred_element_type=jnp.float32)
        o_ref[...] = h2 * dv

    h2 = pl.pallas_call(
        mid_body,
        out_shape=jax.ShapeDtypeStruct((n, h_dim), jnp.float32),
    )(p1, h1, dinv, b1.reshape(1, h_dim), g1.reshape(1, h_dim),
      be1.reshape(1, h_dim), W2)

    p2 = msg(h2, src_p, dst_p)

    bid = b.astype(jnp.int32).reshape(1, n)

    def out_body(p_ref, h_ref, dinv_ref, b2_ref, g2_ref, be2_ref, bid_ref,
                 o_ref):
        dv = dinv_ref[...]
        s = p_ref[0:n, :] + p_ref[n_pad:n_pad + n, :] + h_ref[...]
        o2 = s * dv + b2_ref[...]
        mu = jnp.mean(o2, axis=0, keepdims=True)
        var = jnp.mean((o2 - mu) ** 2, axis=0, keepdims=True)
        y = (o2 - mu) * lax.rsqrt(var + EPS) * g2_ref[...] + be2_ref[...]
        y = jnp.maximum(y, 0.0)
        gi = lax.broadcasted_iota(jnp.int32, (G, n), 0)
        m = (bid_ref[...] == gi).astype(jnp.float32)
        sums = jnp.dot(m, y, preferred_element_type=jnp.float32)
        counts = jnp.sum(m, axis=1, keepdims=True)
        o_ref[...] = sums / jnp.maximum(counts, 1.0)

    out = pl.pallas_call(
        out_body,
        out_shape=jax.ShapeDtypeStruct((G, h_dim), jnp.float32),
    )(p2, h2, dinv, b2.reshape(1, h_dim), g2.reshape(1, h_dim),
      be2.reshape(1, h_dim), bid)

    return out


# speedup vs baseline: 1.0472x; 1.0352x over previous
import functools

import numpy as np
import jax
import jax.numpy as jnp
from jax import lax
from jax.experimental import pallas as pl
from jax.experimental.pallas import tpu as pltpu
from jax.experimental.pallas import tpu_sc as plsc

NC = 2
NS = 16
NW = NC * NS
LANES = 16
CH = 128
NBUF = 8
G = 64
EPS = 1e-5
DEG_W = 16


def _zero_vmem(buf, n_rows, n_cols):
    z = jnp.zeros((LANES,), jnp.float32)
    for i in range(n_rows):
        for j in range(n_cols // LANES):
            buf[i, pl.ds(j * LANES, LANES)] = z


def _fill_ones(buf, n_rows, n_cols):
    o = jnp.ones((LANES,), jnp.float32)
    for i in range(n_rows):
        for j in range(n_cols // LANES):
            buf[i, pl.ds(j * LANES, LANES)] = o


def _make_hist_kernel(n_pad, n_edges):
    rows_per_sub = n_pad // NS
    z_rows = 64
    wave = 6
    g_total = n_edges // CH
    g_full = g_total // NW
    g_rem = g_total - g_full * NW
    n_waves = g_full // wave
    assert g_full % wave == 0
    mesh = plsc.VectorSubcoreMesh(core_axis_name="c", subcore_axis_name="s")

    @functools.partial(
        pl.kernel,
        out_type=jax.ShapeDtypeStruct((2 * n_pad, DEG_W), jnp.float32),
        mesh=mesh,
        scratch_types=[
            pltpu.VMEM_SHARED((n_pad, DEG_W), jnp.float32),
            [pltpu.VMEM((CH,), jnp.int32) for _ in range(wave)],
            pltpu.VMEM((CH,), jnp.int32),
            pltpu.VMEM((CH, DEG_W), jnp.float32),
            pltpu.VMEM((z_rows, DEG_W), jnp.float32),
            [pltpu.SemaphoreType.DMA for _ in range(wave)],
            [pltpu.SemaphoreType.DMA for _ in range(wave)],
        ],
        compiler_params=pltpu.CompilerParams(use_tc_tiling_on_sc=True),
    )
    def k(e_hbm, out_hbm, acc, didx, didx_x, ones, zbuf, sem_i, sem_s):
        c = lax.axis_index("c")
        s = lax.axis_index("s")
        wid = s * NC + c
        _fill_ones(ones, CH, DEG_W)
        _zero_vmem(zbuf, z_rows, DEG_W)
        r0 = s * rows_per_sub
        for t in range(rows_per_sub // z_rows):
            pltpu.sync_copy(zbuf, acc.at[pl.ds(r0 + t * z_rows, z_rows)])
        plsc.subcore_barrier()

        def body(i, carry):
            for j in range(wave):
                t = i * wave + j

                @pl.when(i > 0)
                def _drain():
                    pltpu.make_async_copy(ones, acc.at[didx[j]],
                                          sem_s[j]).wait()

                g = wid + NW * t
                pltpu.async_copy(e_hbm.at[1, pl.ds(g * CH, CH)], didx[j],
                                 sem_i[j])
            for j in range(wave):
                pltpu.make_async_copy(e_hbm.at[1, pl.ds(0, CH)], didx[j],
                                      sem_i[j]).wait()
                pltpu.async_copy(ones, acc.at[didx[j]], sem_s[j], add=True)
            return carry

        lax.fori_loop(0, n_waves, body, 0)
        for j in range(wave):
            pltpu.make_async_copy(ones, acc.at[didx[j]], sem_s[j]).wait()

        @pl.when(wid < g_rem)
        def _tail():
            g = g_full * NW + wid
            pltpu.sync_copy(e_hbm.at[1, pl.ds(g * CH, CH)], didx_x)
            pltpu.sync_copy(ones, acc.at[didx_x], add=True)

        plsc.subcore_barrier()
        out0 = c * n_pad + s * rows_per_sub
        pltpu.sync_copy(acc.at[pl.ds(r0, rows_per_sub)],
                        out_hbm.at[pl.ds(out0, rows_per_sub)])

    return k


def _make_msg_kernel(n_pad, n_chunks, h_dim):
    rows_per_sub = n_pad // NS
    z_rows = 64
    mesh = plsc.VectorSubcoreMesh(core_axis_name="c", subcore_axis_name="s")

    @functools.partial(
        pl.kernel,
        out_type=jax.ShapeDtypeStruct((2 * n_pad, h_dim), jnp.float32),
        mesh=mesh,
        scratch_types=[
            pltpu.VMEM_SHARED((n_pad, h_dim), jnp.float32),
            pltpu.VMEM((n_chunks, CH), jnp.int32),
            pltpu.VMEM((n_chunks, CH), jnp.int32),
            [pltpu.VMEM((CH, h_dim), jnp.float32) for _ in range(NBUF)],
            pltpu.VMEM((z_rows, h_dim), jnp.float32),
            [pltpu.SemaphoreType.DMA for _ in range(NBUF)],
            [pltpu.SemaphoreType.DMA for _ in range(NBUF)],
        ],
        compiler_params=pltpu.CompilerParams(use_tc_tiling_on_sc=False),
    )
    def k(h_hbm, src_hbm, dst_hbm, out_hbm, acc, sidx, didx, rows, zbuf,
          sem_g, sem_s):
        c = lax.axis_index("c")
        s = lax.axis_index("s")
        wid = s * NC + c
        pltpu.sync_copy(src_hbm.at[wid], sidx)
        pltpu.sync_copy(dst_hbm.at[wid], didx)
        _zero_vmem(zbuf, z_rows, h_dim)
        r0 = s * rows_per_sub
        for t in range(rows_per_sub // z_rows):
            pltpu.sync_copy(zbuf, acc.at[pl.ds(r0 + t * z_rows, z_rows)])
        plsc.subcore_barrier()

        def body(i, carry):
            gathers = []
            for j in range(NBUF):
                t = i * NBUF + j

                @pl.when(i > 0)
                def _drain():
                    pltpu.make_async_copy(rows[j], acc.at[didx.at[t]],
                                          sem_s[j]).wait()

                gathers.append(
                    pltpu.async_copy(h_hbm.at[sidx.at[t]], rows[j],
                                     sem_g[j]))
            for j in range(NBUF):
                t = i * NBUF + j
                gathers[j].wait()
                pltpu.async_copy(rows[j], acc.at[didx.at[t]], sem_s[j],
                                 add=True)
            return carry

        n_iter = n_chunks // NBUF
        lax.fori_loop(0, n_iter, body, 0)
        for j in range(NBUF):
            t = (n_iter - 1) * NBUF + j
            pltpu.make_async_copy(rows[j], acc.at[didx.at[t]],
                                  sem_s[j]).wait()
        plsc.subcore_barrier()
        out0 = c * n_pad + s * rows_per_sub
        pltpu.sync_copy(acc.at[pl.ds(r0, rows_per_sub)],
                        out_hbm.at[pl.ds(out0, rows_per_sub)])

    return k


def kernel(x, e, b, W1, b1, g1, be1, W2, b2, g2, be2):
    n, d = x.shape
    h_dim = W1.shape[1]
    n_edges = e.shape[1]

    n_pad = ((n + NW * 64) // (NW * 64)) * (NW * 64)
    grp = NW * CH * NBUF
    ep = ((n_edges + grp - 1) // grp) * grp
    n_chunks = ep // (NW * CH)

    e32 = e.astype(jnp.int32)
    ef = e32.reshape(2 * n_edges)
    src = ef[:n_edges]
    dst = ef[n_edges:]
    pad = ep - n_edges
    pad_np = np.arange(pad, dtype=np.int32)
    src_p = jnp.concatenate([src, jnp.asarray(pad_np % n)])
    dst_p = jnp.concatenate([dst, jnp.asarray(n + pad_np % (n_pad - n))])
    src_p = src_p.reshape(NW, n_chunks, CH)
    dst_p = dst_p.reshape(NW, n_chunks, CH)

    def mm1_body(x_ref, w_ref, o_ref):
        o_ref[...] = jnp.dot(x_ref[...], w_ref[...],
                             preferred_element_type=jnp.float32)

    h1_raw = pl.pallas_call(
        mm1_body,
        out_shape=jax.ShapeDtypeStruct((n, h_dim), jnp.float32),
    )(x, W1)

    hist = _make_hist_kernel(n_pad, n_edges)(e32)

    def scale_body(hist_ref, h_ref, o_ref, dinv_ref):
        deg = hist_ref[0:n, :] + hist_ref[n_pad:n_pad + n, :] + 1.0
        dv = lax.rsqrt(deg)[:, 0:1]
        dv64 = jnp.broadcast_to(dv, (n, h_dim))
        dinv_ref[...] = dv64
        o_ref[...] = h_ref[...] * dv64

    h1, dinv = pl.pallas_call(
        scale_body,
        out_shape=[jax.ShapeDtypeStruct((n, h_dim), jnp.float32),
                   jax.ShapeDtypeStruct((n, h_dim), jnp.float32)],
    )(hist, h1_raw)

    msg = _make_msg_kernel(n_pad, n_chunks, h_dim)

    p1 = msg(h1, src_p, dst_p)

    def mid_body(p_ref, h_ref, dinv_ref, b1_ref, g1_ref, be1_ref, w2_ref,
                 o_ref):
        dv = dinv_ref[...]
        s = p_ref[0:n, :] + p_ref[n_pad:n_pad + n, :] + h_ref[...]
        o1 = s * dv + b1_ref[...]
        mu = jnp.mean(o1, axis=0, keepdims=True)
        var = jnp.mean((o1 - mu) ** 2, axis=0, keepdims=True)
        y = (o1 - mu) * lax.rsqrt(var + EPS) * g1_ref[...] + be1_ref[...]
        y = jnp.maximum(y, 0.0)
        h2 = jnp.dot(y, w2_ref[...], preferred_element_type=jnp.float32)
        o_ref[...] = h2 * dv

    h2 = pl.pallas_call(
        mid_body,
        out_shape=jax.ShapeDtypeStruct((n, h_dim), jnp.float32),
    )(p1, h1, dinv, b1.reshape(1, h_dim), g1.reshape(1, h_dim),
      be1.reshape(1, h_dim), W2)

    p2 = msg(h2, src_p, dst_p)

    bid = b.astype(jnp.int32).reshape(1, n)

    def out_body(p_ref, h_ref, dinv_ref, b2_ref, g2_ref, be2_ref, bid_ref,
                 o_ref):
        dv = dinv_ref[...]
        s = p_ref[0:n, :] + p_ref[n_pad:n_pad + n, :] + h_ref[...]
        o2 = s * dv + b2_ref[...]
        mu = jnp.mean(o2, axis=0, keepdims=True)
        var = jnp.mean((o2 - mu) ** 2, axis=0, keepdims=True)
        y = (o2 - mu) * lax.rsqrt(var + EPS) * g2_ref[...] + be2_ref[...]
        y = jnp.maximum(y, 0.0)
        gi = lax.broadcasted_iota(jnp.int32, (G, n), 0)
        m = (bid_ref[...] == gi).astype(jnp.float32)
        sums = jnp.dot(m, y, preferred_element_type=jnp.float32)
        counts = jnp.sum(m, axis=1, keepdims=True)
        o_ref[...] = sums / jnp.maximum(counts, 1.0)

    out = pl.pallas_call(
        out_body,
        out_shape=jax.ShapeDtypeStruct((G, h_dim), jnp.float32),
    )(p2, h2, dinv, b2.reshape(1, h_dim), g2.reshape(1, h_dim),
      be2.reshape(1, h_dim), bid)

    return out
